# baseline (speedup 1.0000x reference)
import jax
import jax.numpy as jnp
from jax.experimental import pallas as pl

_N = 256
_BB_MIN = -2.0
_BB_MAX = 2.0
_VOXEL_SIZE = (_BB_MAX - _BB_MIN) / (_N - 1)
_INV2VS = 1.0 / (2.0 * _VOXEL_SIZE)

_BX = 8
_NUM_BLOCKS = _N // _BX


def _normals_body(c_ref, ph_ref, nh_ref, o_ref):
    c = c_ref[...]
    inv = jnp.float32(_INV2VS)

    z_int = c[:, :, 2:] - c[:, :, :-2]
    z0 = c[:, :, 1:2] - 1.5 * c[:, :, 0:1] + 0.5 * c[:, :, 2:3]
    zn = 1.5 * c[:, :, -1:] - c[:, :, -2:-1] - 0.5 * c[:, :, -3:-2]
    o_ref[2] = jnp.concatenate([z0, z_int, zn], axis=2) * inv

    y_int = c[:, 2:, :] - c[:, :-2, :]
    y0 = c[:, 1:2, :] - 1.5 * c[:, 0:1, :] + 0.5 * c[:, 2:3, :]
    yn = 1.5 * c[:, -1:, :] - c[:, -2:-1, :] - 0.5 * c[:, -3:-2, :]
    o_ref[1] = jnp.concatenate([y0, y_int, yn], axis=1) * inv

    ph = ph_ref[...]
    nh = nh_ref[...]
    x_p = jnp.concatenate([c[1:], nh], axis=0)
    x_m = jnp.concatenate([ph, c[:-1]], axis=0)
    o_ref[0] = (x_p - x_m) * inv

    i = pl.program_id(0)

    @pl.when(i == 0)
    def _fix_first():
        o_ref[0, 0] = (c[1] - 1.5 * c[0] + 0.5 * c[2]) * inv

    @pl.when(i == _NUM_BLOCKS - 1)
    def _fix_last():
        o_ref[0, _BX - 1] = (
            1.5 * c[_BX - 1] - c[_BX - 2] - 0.5 * c[_BX - 3]
        ) * inv


def kernel(grid):
    return pl.pallas_call(
        _normals_body,
        grid=(_NUM_BLOCKS,),
        in_specs=[
            pl.BlockSpec((_BX, _N, _N), lambda i: (i, 0, 0)),
            pl.BlockSpec(
                (1, _N, _N), lambda i: (jnp.maximum(i * _BX - 1, 0), 0, 0)
            ),
            pl.BlockSpec(
                (1, _N, _N),
                lambda i: (jnp.minimum(i * _BX + _BX, _N - 1), 0, 0),
            ),
        ],
        out_specs=pl.BlockSpec((3, _BX, _N, _N), lambda i: (0, i, 0, 0)),
        out_shape=jax.ShapeDtypeStruct((3, _N, _N, _N), jnp.float32),
    )(grid, grid, grid)


# speedup vs baseline: 1.1236x; 1.1236x over previous
import jax
import jax.numpy as jnp
from jax.experimental import pallas as pl

_N = 256
_BB_MIN = -2.0
_BB_MAX = 2.0
_VOXEL_SIZE = (_BB_MAX - _BB_MIN) / (_N - 1)
_INV2VS = 1.0 / (2.0 * _VOXEL_SIZE)

_BX = 16
_NUM_BLOCKS = _N // _BX


def _normals_body(c_ref, ph_ref, nh_ref, o_ref):
    c = c_ref[...]
    inv = jnp.float32(_INV2VS)

    z_int = c[:, :, 2:] - c[:, :, :-2]
    z0 = c[:, :, 1:2] - 1.5 * c[:, :, 0:1] + 0.5 * c[:, :, 2:3]
    zn = 1.5 * c[:, :, -1:] - c[:, :, -2:-1] - 0.5 * c[:, :, -3:-2]
    o_ref[2] = jnp.concatenate([z0, z_int, zn], axis=2) * inv

    y_int = c[:, 2:, :] - c[:, :-2, :]
    y0 = c[:, 1:2, :] - 1.5 * c[:, 0:1, :] + 0.5 * c[:, 2:3, :]
    yn = 1.5 * c[:, -1:, :] - c[:, -2:-1, :] - 0.5 * c[:, -3:-2, :]
    o_ref[1] = jnp.concatenate([y0, y_int, yn], axis=1) * inv

    ph = ph_ref[...]
    nh = nh_ref[...]
    x_p = jnp.concatenate([c[1:], nh], axis=0)
    x_m = jnp.concatenate([ph, c[:-1]], axis=0)
    o_ref[0] = (x_p - x_m) * inv

    i = pl.program_id(0)

    @pl.when(i == 0)
    def _fix_first():
        o_ref[0, 0] = (c[1] - 1.5 * c[0] + 0.5 * c[2]) * inv

    @pl.when(i == _NUM_BLOCKS - 1)
    def _fix_last():
        o_ref[0, _BX - 1] = (
            1.5 * c[_BX - 1] - c[_BX - 2] - 0.5 * c[_BX - 3]
        ) * inv


def kernel(grid):
    return pl.pallas_call(
        _normals_body,
        grid=(_NUM_BLOCKS,),
        in_specs=[
            pl.BlockSpec((_BX, _N, _N), lambda i: (i, 0, 0)),
            pl.BlockSpec(
                (1, _N, _N), lambda i: (jnp.maximum(i * _BX - 1, 0), 0, 0)
            ),
            pl.BlockSpec(
                (1, _N, _N),
                lambda i: (jnp.minimum(i * _BX + _BX, _N - 1), 0, 0),
            ),
        ],
        out_specs=pl.BlockSpec((3, _BX, _N, _N), lambda i: (0, i, 0, 0)),
        out_shape=jax.ShapeDtypeStruct((3, _N, _N, _N), jnp.float32),
    )(grid, grid, grid)
